# baseline (speedup 1.0000x reference)
import functools

import jax
import jax.numpy as jnp
from jax import lax
from jax.experimental import pallas as pl
from jax.experimental.pallas import tpu as pltpu
from jax.experimental.pallas import tpu_sc as plsc

_MAX_NORM = 2.0

_L = 16
_NC = 2
_NS = 16
_NW = _NC * _NS
_B = 16384
_BPW = _B // _NW
_CD = 16
_FD = 48
_OD = _CD + _FD
_GROUPS = _BPW // _L


def _rsqrt_newton(x):
    i = lax.bitcast_convert_type(x, jnp.int32)
    i = jnp.int32(0x5F3759DF) - lax.shift_right_logical(i, 1)
    y = lax.bitcast_convert_type(i, jnp.float32)
    for _ in range(3):
        y = y * (1.5 - 0.5 * x * y * y)
    return y


def _scale_from_ss(ss):
    return jnp.minimum(1.0, _MAX_NORM * _rsqrt_newton(jnp.maximum(ss, 1e-14)))


def _make_sc_kernel():
    mesh = plsc.VectorSubcoreMesh(core_axis_name="c", subcore_axis_name="s")

    @functools.partial(
        pl.kernel,
        mesh=mesh,
        out_type=jax.ShapeDtypeStruct((_B, _OD), jnp.float32),
        compiler_params=pltpu.CompilerParams(
            needs_layout_passes=False, use_tc_tiling_on_sc=False),
        scratch_types=[
            pltpu.VMEM((_BPW,), jnp.int32),
            pltpu.VMEM((_BPW,), jnp.int32),
            pltpu.VMEM((_BPW, _CD), jnp.float32),
            pltpu.VMEM((_BPW, _FD), jnp.float32),
            pltpu.VMEM((_BPW, _OD), jnp.float32),
            pltpu.SemaphoreType.DMA,
            pltpu.SemaphoreType.DMA,
        ],
    )
    def run(ct_hbm, ft_hbm, ci_hbm, fi_hbm, out_hbm,
            cidx_v, fidx_v, crows_v, frows_v, ostage_v, csem, fsem):
        wid = lax.axis_index("s") * _NC + lax.axis_index("c")
        base = wid * _BPW

        pltpu.sync_copy(ci_hbm.at[pl.ds(base, _BPW)], cidx_v)
        pltpu.sync_copy(fi_hbm.at[pl.ds(base, _BPW)], fidx_v)

        ccopy = pltpu.async_copy(ct_hbm.at[cidx_v], crows_v, csem)
        fcopy = pltpu.async_copy(ft_hbm.at[fidx_v], frows_v, fsem)
        ccopy.wait()
        fcopy.wait()

        lanes = lax.iota(jnp.int32, _L)

        def group_body(g, carry):
            rows = g * _L + lanes

            css = jnp.zeros((_L,), jnp.float32)
            for c in range(_CD):
                col = jnp.full((_L,), c, jnp.int32)
                v = plsc.load_gather(crows_v, [rows, col])
                css = css + v * v
            fss = jnp.zeros((_L,), jnp.float32)
            for c in range(_FD):
                col = jnp.full((_L,), c, jnp.int32)
                v = plsc.load_gather(frows_v, [rows, col])
                fss = fss + v * v

            cscale = _scale_from_ss(css)
            fscale = _scale_from_ss(fss)

            for c in range(_CD):
                col = jnp.full((_L,), c, jnp.int32)
                v = plsc.load_gather(crows_v, [rows, col])
                plsc.store_scatter(ostage_v, [rows, col], v * cscale)
            for c in range(_FD):
                col = jnp.full((_L,), c, jnp.int32)
                v = plsc.load_gather(frows_v, [rows, col])
                ocol = jnp.full((_L,), c + _CD, jnp.int32)
                plsc.store_scatter(ostage_v, [rows, ocol], v * fscale)
            return carry

        lax.fori_loop(0, _GROUPS, group_body, 0)

        pltpu.sync_copy(ostage_v, out_hbm.at[pl.ds(base, _BPW)])

    return run


_sc_kernel = _make_sc_kernel()


@jax.jit
def kernel(class_table, function_table, class_indices, function_indices):
    ci = class_indices.astype(jnp.int32)
    fi = function_indices.astype(jnp.int32)
    return _sc_kernel(class_table, function_table, ci, fi)


# speedup vs baseline: 4.9300x; 4.9300x over previous
import functools

import jax
import jax.numpy as jnp
from jax import lax
from jax.experimental import pallas as pl
from jax.experimental.pallas import tpu as pltpu
from jax.experimental.pallas import tpu_sc as plsc

_MAX_NORM = 2.0

_L = 16
_NC = 2
_NS = 16
_NW = _NC * _NS
_B = 16384
_BPW = _B // _NW
_CD = 16
_FD = 48
_OD = _CD + _FD
_CH = 16
_NCHUNK = _BPW // _CH
_GPC = _CH // _L

_CV = 100000
_FV = 1000000
_CT_ROWS = _CV * _CD // 128


def _rsqrt_newton(x):
    i = lax.bitcast_convert_type(x, jnp.int32)
    i = jnp.int32(0x5F3759DF) - lax.shift_right_logical(i, 1)
    y = lax.bitcast_convert_type(i, jnp.float32)
    for _ in range(3):
        y = y * (1.5 - 0.5 * x * y * y)
    return y


def _scale_from_ss(ss):
    return jnp.minimum(1.0, _MAX_NORM * _rsqrt_newton(jnp.maximum(ss, 1e-14)))


def _make_sc_kernel():
    mesh = plsc.VectorSubcoreMesh(core_axis_name="c", subcore_axis_name="s")

    @functools.partial(
        pl.kernel,
        mesh=mesh,
        out_type=jax.ShapeDtypeStruct((_B * _OD,), jnp.float32),
        compiler_params=pltpu.CompilerParams(needs_layout_passes=False),
        scratch_types=[
            pltpu.VMEM((_BPW,), jnp.int32),
            pltpu.VMEM((_BPW,), jnp.int32),
            pltpu.VMEM((_BPW,), jnp.int32),
            pltpu.VMEM((_CH, 128), jnp.float32),
            pltpu.VMEM((_CH, _FD, 128), jnp.float32),
            pltpu.VMEM((_CH * _OD,), jnp.float32),
            pltpu.SemaphoreType.DMA,
            pltpu.SemaphoreType.DMA,
        ],
    )
    def run(ct_hbm, ft_hbm, ci_hbm, fi_hbm, out_hbm,
            cidx_v, fidx_v, cblk_v, cbuf_v, fbuf_v, ostage_v, csem, fsem):
        wid = lax.axis_index("s") * _NC + lax.axis_index("c")
        base = wid * _BPW

        pltpu.sync_copy(ci_hbm.at[pl.ds(base, _BPW)], cidx_v)
        pltpu.sync_copy(fi_hbm.at[pl.ds(base, _BPW)], fidx_v)

        def blk_body(i, carry):
            civ = cidx_v[pl.ds(i * _L, _L)]
            cblk_v[pl.ds(i * _L, _L)] = lax.shift_right_logical(civ, 3)
            return carry

        lax.fori_loop(0, _BPW // _L, blk_body, 0)

        lanes = lax.iota(jnp.int32, _L)

        def chunk_body(c, carry):
            c0 = c * _CH

            ccopy = pltpu.async_copy(
                ct_hbm.at[cblk_v.at[pl.ds(c0, _CH)]], cbuf_v, csem)

            fiv_issue = fidx_v[pl.ds(c0, _L)]
            for k in range(_L):
                col128 = pl.multiple_of(
                    lax.shift_left(lax.shift_right_logical(fiv_issue[k], 7), 7),
                    128)
                pltpu.async_copy(
                    ft_hbm.at[:, pl.ds(col128, 128)], fbuf_v.at[k], fsem)

            ccopy.wait()

            def drain_body(k, carry):
                pltpu.make_async_copy(
                    ft_hbm.at[:, pl.ds(0, 128)], fbuf_v.at[k], fsem).wait()
                return carry

            lax.fori_loop(0, _CH, drain_body, 0)

            rows16 = lanes
            civ = cidx_v[pl.ds(c0, _L)]
            fiv = fidx_v[pl.ds(c0, _L)]
            ccol0 = lax.shift_left(jnp.bitwise_and(civ, 7), 4)
            fcol = jnp.bitwise_and(fiv, 127)

            css = jnp.zeros((_L,), jnp.float32)
            for j in range(_CD):
                v = plsc.load_gather(cbuf_v, [rows16, ccol0 + j])
                css = css + v * v
            fss = jnp.zeros((_L,), jnp.float32)
            for d in range(_FD):
                v = plsc.load_gather(
                    fbuf_v, [rows16, jnp.full((_L,), d, jnp.int32), fcol])
                fss = fss + v * v

            cscale = _scale_from_ss(css)
            fscale = _scale_from_ss(fss)

            obase = rows16 * _OD
            for j in range(_CD):
                v = plsc.load_gather(cbuf_v, [rows16, ccol0 + j])
                plsc.store_scatter(ostage_v, [obase + j], v * cscale)
            for d in range(_FD):
                v = plsc.load_gather(
                    fbuf_v, [rows16, jnp.full((_L,), d, jnp.int32), fcol])
                plsc.store_scatter(ostage_v, [obase + _CD + d], v * fscale)

            pltpu.sync_copy(
                ostage_v, out_hbm.at[pl.ds((base + c0) * _OD, _CH * _OD)])
            return carry

        lax.fori_loop(0, _NCHUNK, chunk_body, 0)

    return run


_sc_kernel = _make_sc_kernel()


@jax.jit
def kernel(class_table, function_table, class_indices, function_indices):
    ci = class_indices.astype(jnp.int32)
    fi = function_indices.astype(jnp.int32)
    ct128 = class_table.reshape(_CT_ROWS, 128)
    ftT = function_table.T
    flat = _sc_kernel(ct128, ftT, ci, fi)
    return flat.reshape(_B, _OD)
